# baseline (speedup 1.0000x reference)
import functools

import jax
import jax.numpy as jnp
from jax import lax
from jax.experimental import pallas as pl
from jax.experimental.pallas import tpu as pltpu
from jax.experimental.pallas import tpu_sc as plsc

N = 10000
E = 160000
D = 256
HALF = 128
DL = 128

NS = 16
EPT = E // NS
CH = 80
NCHUNK = EPT // CH
NPT = 624
DEG_PAD = 640
NPAD = NS * DEG_PAD

_mesh = plsc.VectorSubcoreMesh(core_axis_name="c", subcore_axis_name="s")


@functools.partial(
    pl.kernel,
    out_type=jax.ShapeDtypeStruct((NS, DEG_PAD), jnp.float32),
    mesh=_mesh,
    scratch_types=[
        pltpu.VMEM((EPT,), jnp.int32),
        pltpu.VMEM((NPAD,), jnp.float32),
        pltpu.VMEM((NS, DEG_PAD), jnp.float32),
        pltpu.VMEM_SHARED((NS, NPAD), jnp.float32),
    ],
    compiler_params=pltpu.CompilerParams(needs_layout_passes=False),
)
def _deg_kernel(dst_hbm, out, dst_v, acc_v, part_v, shared):
    c = lax.axis_index("c")
    s = lax.axis_index("s")

    @pl.when(c == 0)
    def _():
        zero16 = jnp.zeros((16,), jnp.float32)

        def zb(i, carry):
            acc_v[pl.ds(i * 16, 16)] = zero16
            return carry

        lax.fori_loop(0, NPAD // 16, zb, 0)

        pltpu.sync_copy(dst_hbm.at[s], dst_v)
        ones = jnp.ones((16,), jnp.float32)

        def body(i, carry):
            idx = dst_v[pl.ds(i * 16, 16)]
            plsc.addupdate_scatter(acc_v, [idx], ones)
            return carry

        lax.fori_loop(0, EPT // 16, body, 0)

        pltpu.sync_copy(acc_v, shared.at[s])
        plsc.subcore_barrier()
        pltpu.sync_copy(shared.at[:, pl.ds(s * DEG_PAD, DEG_PAD)], part_v)

        def red(j, carry):
            v = part_v[0, pl.ds(j * 16, 16)]
            for t in range(1, NS):
                v = v + part_v[t, pl.ds(j * 16, 16)]
            acc_v[pl.ds(j * 16, 16)] = v
            return carry

        lax.fori_loop(0, DEG_PAD // 16, red, 0)
        pltpu.sync_copy(acc_v.at[pl.ds(0, DEG_PAD)], out.at[s])


@functools.partial(
    pl.kernel,
    out_type=(
        jax.ShapeDtypeStruct((N, HALF), jnp.float32),
        jax.ShapeDtypeStruct((N, HALF), jnp.float32),
    ),
    mesh=_mesh,
    scratch_types=[
        [pltpu.VMEM((CH,), jnp.int32)] * 3,
        [pltpu.VMEM((CH,), jnp.int32)] * 3,
        [pltpu.VMEM((CH,), jnp.int32)] * 3,
        [pltpu.VMEM((CH, HALF), jnp.float32)] * 3,
        pltpu.VMEM_SHARED((N, HALF), jnp.float32),
        [pltpu.SemaphoreType.DMA] * 3,
        [pltpu.SemaphoreType.DMA] * 3,
        [pltpu.SemaphoreType.DMA] * 3,
    ],
    compiler_params=pltpu.CompilerParams(needs_layout_passes=False),
)
def _agg_kernel(edges_hbm, pa, pb, out_a, out_b,
                pidx, sidx, didx, buf, acc_sh, sem_i, sem_g, sem_s):
    c = lax.axis_index("c")
    s = lax.axis_index("s")

    zero16 = jnp.zeros((16,), jnp.float32)

    def zb(i, carry):
        for j in range(HALF // 16):
            buf[0][i, pl.ds(j * 16, 16)] = zero16
        return carry

    lax.fori_loop(0, CH, zb, 0)
    for j in range(NPT // CH):
        base = pl.multiple_of(s * NPT + j * CH, 8)
        pltpu.sync_copy(buf[0], acc_sh.at[pl.ds(base, CH)])
    rem = NPT - (NPT // CH) * CH
    if rem:
        base = pl.multiple_of(s * NPT + NPT - rem, 8)
        pltpu.sync_copy(buf[0].at[pl.ds(0, rem)], acc_sh.at[pl.ds(base, rem)])

    @pl.when(s == NS - 1)
    def _():
        pltpu.sync_copy(buf[0].at[pl.ds(0, N - NS * NPT)],
                        acc_sh.at[pl.ds(NS * NPT, N - NS * NPT)])

    def i_issue(ch, i):
        pltpu.async_copy(edges_hbm.at[s * NCHUNK + ch], pidx[i], sem_i[i])

    def i_wait(ch, i):
        pltpu.make_async_copy(edges_hbm.at[s * NCHUNK + ch], pidx[i],
                              sem_i[i]).wait()

    def unpack_s(i):
        for v in range(CH // 16):
            pk = pidx[i][pl.ds(v * 16, 16)]
            sidx[i][pl.ds(v * 16, 16)] = lax.shift_right_logical(pk, 16)

    def unpack_d(i):
        for v in range(CH // 16):
            pk = pidx[i][pl.ds(v * 16, 16)]
            didx[i][pl.ds(v * 16, 16)] = lax.bitwise_and(pk, 0xFFFF)

    plsc.subcore_barrier()

    def run(p_ref, out_ref):
        def g_issue(i):
            pltpu.async_copy(p_ref.at[sidx[i]], buf[i], sem_g[i])

        def g_wait(i):
            pltpu.make_async_copy(p_ref.at[sidx[i]], buf[i], sem_g[i]).wait()

        def s_issue(i):
            pltpu.async_copy(buf[i], acc_sh.at[didx[i]], sem_s[i], add=True)

        def s_wait(i):
            pltpu.make_async_copy(buf[i], acc_sh.at[didx[i]], sem_s[i]).wait()

        for i in range(3):
            i_issue(i, i)
        for i in range(3):
            i_wait(i, i)
            unpack_s(i)
            unpack_d(i)
            g_issue(i)
            i_issue(i + 3, i)

        def body(k, carry):
            c0 = 3 * k
            for i in range(3):
                ch = c0 + i
                g_wait(i)
                s_issue(i)

                @pl.when(ch + 3 < NCHUNK)
                def _():
                    i_wait(ch + 3, i)
                    unpack_s(i)
            for i in range(3):
                ch = c0 + i
                cn = ch + 3

                @pl.when(cn < NCHUNK)
                def _():
                    s_wait(i)
                    g_issue(i)
                    unpack_d(i)

                    @pl.when(cn + 3 < NCHUNK)
                    def _():
                        i_issue(cn + 3, i)
            return carry

        lax.fori_loop(0, NCHUNK // 3, body, 0)
        for i in range(NCHUNK - 3 * (NCHUNK // 3)):
            g_wait(i)
            s_issue(i)
        for i in range(3):
            s_wait(i)
        plsc.subcore_barrier()
        base = pl.multiple_of(s * NPT, 8)
        pltpu.sync_copy(acc_sh.at[pl.ds(base, NPT)],
                        out_ref.at[pl.ds(base, NPT)])

        @pl.when(s == NS - 1)
        def _():
            pltpu.sync_copy(acc_sh.at[pl.ds(NS * NPT, N - NS * NPT)],
                            out_ref.at[pl.ds(NS * NPT, N - NS * NPT)])

    @pl.when(c == 0)
    def _():
        run(pa, out_a)

    @pl.when(c == 1)
    def _():
        run(pb, out_b)


ROWS = 1000
GRID = N // ROWS


def _scale_body(deg_ref, y_ref, pa_ref, pb_ref):
    dinv = lax.rsqrt(deg_ref[...] + 1.0)
    p = dinv * y_ref[...]
    pa_ref[...] = p[:, :HALF]
    pb_ref[...] = p[:, HALF:]


_scale_call = pl.pallas_call(
    _scale_body,
    grid=(GRID,),
    in_specs=[
        pl.BlockSpec((ROWS, 1), lambda i: (i, 0)),
        pl.BlockSpec((ROWS, D), lambda i: (i, 0)),
    ],
    out_specs=[
        pl.BlockSpec((ROWS, HALF), lambda i: (i, 0)),
        pl.BlockSpec((ROWS, HALF), lambda i: (i, 0)),
    ],
    out_shape=[
        jax.ShapeDtypeStruct((N, HALF), jnp.float32),
        jax.ShapeDtypeStruct((N, HALF), jnp.float32),
    ],
)


def _mid_body(deg_ref, aa, ab, pa, pb, w_ref, b_ref, oa, ob):
    dinv = lax.rsqrt(deg_ref[...] + 1.0)
    agg = jnp.concatenate([aa[...] + pa[...], ab[...] + pb[...]], axis=1) * dinv
    h = jnp.dot(agg, w_ref[...], preferred_element_type=jnp.float32) + b_ref[...]
    p2 = dinv * jnp.maximum(h, 0.0)
    oa[...] = p2[:, :HALF]
    ob[...] = p2[:, HALF:]


_mid_call = pl.pallas_call(
    _mid_body,
    grid=(GRID,),
    in_specs=[
        pl.BlockSpec((ROWS, 1), lambda i: (i, 0)),
        pl.BlockSpec((ROWS, HALF), lambda i: (i, 0)),
        pl.BlockSpec((ROWS, HALF), lambda i: (i, 0)),
        pl.BlockSpec((ROWS, HALF), lambda i: (i, 0)),
        pl.BlockSpec((ROWS, HALF), lambda i: (i, 0)),
        pl.BlockSpec((D, D), lambda i: (0, 0)),
        pl.BlockSpec((1, D), lambda i: (0, 0)),
    ],
    out_specs=[
        pl.BlockSpec((ROWS, HALF), lambda i: (i, 0)),
        pl.BlockSpec((ROWS, HALF), lambda i: (i, 0)),
    ],
    out_shape=[
        jax.ShapeDtypeStruct((N, HALF), jnp.float32),
        jax.ShapeDtypeStruct((N, HALF), jnp.float32),
    ],
)


def _final_body(deg_ref, aa, ab, pa, pb, wmu_ref, bmu_ref, wlv_ref, blv_ref,
                mu_ref, lv_ref):
    dinv = lax.rsqrt(deg_ref[...] + 1.0)
    agg = jnp.concatenate([aa[...] + pa[...], ab[...] + pb[...]], axis=1) * dinv
    mu_ref[...] = jnp.dot(agg, wmu_ref[...],
                          preferred_element_type=jnp.float32) + bmu_ref[...]
    lv = jnp.dot(agg, wlv_ref[...],
                 preferred_element_type=jnp.float32) + blv_ref[...]
    lv_ref[...] = jnp.clip(lv, -10.0, 10.0)


_final_call = pl.pallas_call(
    _final_body,
    grid=(GRID,),
    in_specs=[
        pl.BlockSpec((ROWS, 1), lambda i: (i, 0)),
        pl.BlockSpec((ROWS, HALF), lambda i: (i, 0)),
        pl.BlockSpec((ROWS, HALF), lambda i: (i, 0)),
        pl.BlockSpec((ROWS, HALF), lambda i: (i, 0)),
        pl.BlockSpec((ROWS, HALF), lambda i: (i, 0)),
        pl.BlockSpec((D, DL), lambda i: (0, 0)),
        pl.BlockSpec((1, DL), lambda i: (0, 0)),
        pl.BlockSpec((D, DL), lambda i: (0, 0)),
        pl.BlockSpec((1, DL), lambda i: (0, 0)),
    ],
    out_specs=[
        pl.BlockSpec((ROWS, DL), lambda i: (i, 0)),
        pl.BlockSpec((ROWS, DL), lambda i: (i, 0)),
    ],
    out_shape=[
        jax.ShapeDtypeStruct((N, DL), jnp.float32),
        jax.ShapeDtypeStruct((N, DL), jnp.float32),
    ],
)


def kernel(Y, edge_index, W1, b1, Wmu, bmu, Wlv, blv):
    src = edge_index[0]
    dst = edge_index[1]
    dst16 = dst.reshape(NS, EPT)
    packed = jnp.bitwise_or(jnp.left_shift(src, 16), dst)
    edges_ch = packed.reshape(E // CH, CH)

    degp = _deg_kernel(dst16)
    deg_col = degp.reshape(-1)[:N].reshape(N, 1)

    p1a, p1b = _scale_call(deg_col, Y)
    a1a, a1b = _agg_kernel(edges_ch, p1a, p1b)
    p2a, p2b = _mid_call(deg_col, a1a, a1b, p1a, p1b, W1, b1.reshape(1, D))
    a2a, a2b = _agg_kernel(edges_ch, p2a, p2b)
    mu, lv = _final_call(deg_col, a2a, a2b, p2a, p2b,
                         Wmu, bmu.reshape(1, DL), Wlv, blv.reshape(1, DL))
    return (mu, lv)


# speedup vs baseline: 1.1014x; 1.1014x over previous
import functools

import jax
import jax.numpy as jnp
from jax import lax
from jax.experimental import pallas as pl
from jax.experimental.pallas import tpu as pltpu
from jax.experimental.pallas import tpu_sc as plsc

N = 10000
E = 160000
D = 256
HALF = 128
DL = 128

NS = 16
EPT = E // NS
CH = 128
NCH_STD = 78
NCH_LAST = 80
ZC = 120
NPT = 624
DEG_PAD = 640
NPAD = NS * DEG_PAD

_mesh = plsc.VectorSubcoreMesh(core_axis_name="c", subcore_axis_name="s")


@functools.partial(
    pl.kernel,
    out_type=jax.ShapeDtypeStruct((NS, DEG_PAD), jnp.float32),
    mesh=_mesh,
    scratch_types=[
        pltpu.VMEM((EPT,), jnp.int32),
        pltpu.VMEM((NPAD,), jnp.float32),
        pltpu.VMEM((NS, DEG_PAD), jnp.float32),
        pltpu.VMEM_SHARED((NS, NPAD), jnp.float32),
    ],
    compiler_params=pltpu.CompilerParams(needs_layout_passes=False),
)
def _deg_kernel(dst_hbm, out, dst_v, acc_v, part_v, shared):
    c = lax.axis_index("c")
    s = lax.axis_index("s")

    @pl.when(c == 0)
    def _():
        zero16 = jnp.zeros((16,), jnp.float32)

        def zb(i, carry):
            acc_v[pl.ds(i * 16, 16)] = zero16
            return carry

        lax.fori_loop(0, NPAD // 16, zb, 0)

        pltpu.sync_copy(dst_hbm.at[s], dst_v)
        ones = jnp.ones((16,), jnp.float32)

        def body(i, carry):
            idx = dst_v[pl.ds(i * 16, 16)]
            plsc.addupdate_scatter(acc_v, [idx], ones)
            return carry

        lax.fori_loop(0, EPT // 16, body, 0)

        pltpu.sync_copy(acc_v, shared.at[s])
        plsc.subcore_barrier()
        pltpu.sync_copy(shared.at[:, pl.ds(s * DEG_PAD, DEG_PAD)], part_v)

        def red(j, carry):
            v = part_v[0, pl.ds(j * 16, 16)]
            for t in range(1, NS):
                v = v + part_v[t, pl.ds(j * 16, 16)]
            acc_v[pl.ds(j * 16, 16)] = v
            return carry

        lax.fori_loop(0, DEG_PAD // 16, red, 0)
        pltpu.sync_copy(acc_v.at[pl.ds(0, DEG_PAD)], out.at[s])


@functools.partial(
    pl.kernel,
    out_type=(
        jax.ShapeDtypeStruct((N, HALF), jnp.float32),
        jax.ShapeDtypeStruct((N, HALF), jnp.float32),
    ),
    mesh=_mesh,
    scratch_types=[
        [pltpu.VMEM((CH,), jnp.int32)] * 2,
        [pltpu.VMEM((CH,), jnp.int32)] * 2,
        [pltpu.VMEM((CH,), jnp.int32)] * 2,
        [pltpu.VMEM((CH, HALF), jnp.float32)] * 2,
        pltpu.VMEM_SHARED((N, HALF), jnp.float32),
        [pltpu.SemaphoreType.DMA] * 2,
        [pltpu.SemaphoreType.DMA] * 2,
    ],
    compiler_params=pltpu.CompilerParams(needs_layout_passes=False),
)
def _agg_kernel(edges_hbm, pa, pb, out_a, out_b,
                pidx, sidx, didx, buf, acc_sh, sem_i, sem_g):
    c = lax.axis_index("c")
    s = lax.axis_index("s")
    nch = jnp.where(s == NS - 1, NCH_LAST, NCH_STD)

    zero16 = jnp.zeros((16,), jnp.float32)

    def zb(i, carry):
        for j in range(HALF // 16):
            buf[0][i, pl.ds(j * 16, 16)] = zero16
        return carry

    lax.fori_loop(0, ZC, zb, 0)
    for j in range(NPT // ZC):
        base = pl.multiple_of(s * NPT + j * ZC, 8)
        pltpu.sync_copy(buf[0].at[pl.ds(0, ZC)], acc_sh.at[pl.ds(base, ZC)])
    rem = NPT - (NPT // ZC) * ZC
    if rem:
        base = pl.multiple_of(s * NPT + NPT - rem, 8)
        pltpu.sync_copy(buf[0].at[pl.ds(0, rem)], acc_sh.at[pl.ds(base, rem)])

    @pl.when(s == NS - 1)
    def _():
        pltpu.sync_copy(buf[0].at[pl.ds(0, N - NS * NPT)],
                        acc_sh.at[pl.ds(NS * NPT, N - NS * NPT)])

    def i_issue(ch, i):
        pltpu.async_copy(edges_hbm.at[s * NCH_STD + ch], pidx[i], sem_i[i])

    def i_wait(ch, i):
        pltpu.make_async_copy(edges_hbm.at[s * NCH_STD + ch], pidx[i],
                              sem_i[i]).wait()

    def unpack(i):
        for v in range(CH // 16):
            pk = pidx[i][pl.ds(v * 16, 16)]
            sidx[i][pl.ds(v * 16, 16)] = lax.shift_right_logical(pk, 16)
            didx[i][pl.ds(v * 16, 16)] = lax.bitwise_and(pk, 0xFFFF)

    plsc.subcore_barrier()

    def run(p_ref, out_ref):
        def g_issue(i):
            pltpu.async_copy(p_ref.at[sidx[i]], buf[i], sem_g[i])

        def g_wait(i):
            pltpu.make_async_copy(p_ref.at[sidx[i]], buf[i], sem_g[i]).wait()

        def s_sync(i):
            pltpu.sync_copy(buf[i], acc_sh.at[didx[i]], add=True)

        i_issue(0, 0)
        i_issue(1, 1)
        i_wait(0, 0)
        unpack(0)
        g_issue(0)

        def body(k, carry):
            k2 = 2 * k
            i_wait(k2 + 1, 1)
            unpack(1)
            g_issue(1)
            i_issue(k2 + 2, 0)
            g_wait(0)
            s_sync(0)
            i_wait(k2 + 2, 0)
            unpack(0)
            g_issue(0)
            i_issue(k2 + 3, 1)
            g_wait(1)
            s_sync(1)
            return carry

        lax.fori_loop(0, nch // 2 - 1, body, 0)
        i_wait(nch - 1, 1)
        unpack(1)
        g_issue(1)
        g_wait(0)
        s_sync(0)
        g_wait(1)
        s_sync(1)
        plsc.subcore_barrier()
        base = pl.multiple_of(s * NPT, 8)
        pltpu.sync_copy(acc_sh.at[pl.ds(base, NPT)],
                        out_ref.at[pl.ds(base, NPT)])

        @pl.when(s == NS - 1)
        def _():
            pltpu.sync_copy(acc_sh.at[pl.ds(NS * NPT, N - NS * NPT)],
                            out_ref.at[pl.ds(NS * NPT, N - NS * NPT)])

    @pl.when(c == 0)
    def _():
        run(pa, out_a)

    @pl.when(c == 1)
    def _():
        run(pb, out_b)


ROWS = 1000
GRID = N // ROWS


def _scale_body(deg_ref, y_ref, pa_ref, pb_ref):
    dinv = lax.rsqrt(deg_ref[...] + 1.0)
    p = dinv * y_ref[...]
    pa_ref[...] = p[:, :HALF]
    pb_ref[...] = p[:, HALF:]


_scale_call = pl.pallas_call(
    _scale_body,
    grid=(GRID,),
    in_specs=[
        pl.BlockSpec((ROWS, 1), lambda i: (i, 0)),
        pl.BlockSpec((ROWS, D), lambda i: (i, 0)),
    ],
    out_specs=[
        pl.BlockSpec((ROWS, HALF), lambda i: (i, 0)),
        pl.BlockSpec((ROWS, HALF), lambda i: (i, 0)),
    ],
    out_shape=[
        jax.ShapeDtypeStruct((N, HALF), jnp.float32),
        jax.ShapeDtypeStruct((N, HALF), jnp.float32),
    ],
)


def _mid_body(deg_ref, aa, ab, pa, pb, w_ref, b_ref, oa, ob):
    dinv = lax.rsqrt(deg_ref[...] + 1.0)
    agg = jnp.concatenate([aa[...] + pa[...], ab[...] + pb[...]], axis=1) * dinv
    h = jnp.dot(agg, w_ref[...], preferred_element_type=jnp.float32) + b_ref[...]
    p2 = dinv * jnp.maximum(h, 0.0)
    oa[...] = p2[:, :HALF]
    ob[...] = p2[:, HALF:]


_mid_call = pl.pallas_call(
    _mid_body,
    grid=(GRID,),
    in_specs=[
        pl.BlockSpec((ROWS, 1), lambda i: (i, 0)),
        pl.BlockSpec((ROWS, HALF), lambda i: (i, 0)),
        pl.BlockSpec((ROWS, HALF), lambda i: (i, 0)),
        pl.BlockSpec((ROWS, HALF), lambda i: (i, 0)),
        pl.BlockSpec((ROWS, HALF), lambda i: (i, 0)),
        pl.BlockSpec((D, D), lambda i: (0, 0)),
        pl.BlockSpec((1, D), lambda i: (0, 0)),
    ],
    out_specs=[
        pl.BlockSpec((ROWS, HALF), lambda i: (i, 0)),
        pl.BlockSpec((ROWS, HALF), lambda i: (i, 0)),
    ],
    out_shape=[
        jax.ShapeDtypeStruct((N, HALF), jnp.float32),
        jax.ShapeDtypeStruct((N, HALF), jnp.float32),
    ],
)


def _final_body(deg_ref, aa, ab, pa, pb, wmu_ref, bmu_ref, wlv_ref, blv_ref,
                mu_ref, lv_ref):
    dinv = lax.rsqrt(deg_ref[...] + 1.0)
    agg = jnp.concatenate([aa[...] + pa[...], ab[...] + pb[...]], axis=1) * dinv
    mu_ref[...] = jnp.dot(agg, wmu_ref[...],
                          preferred_element_type=jnp.float32) + bmu_ref[...]
    lv = jnp.dot(agg, wlv_ref[...],
                 preferred_element_type=jnp.float32) + blv_ref[...]
    lv_ref[...] = jnp.clip(lv, -10.0, 10.0)


_final_call = pl.pallas_call(
    _final_body,
    grid=(GRID,),
    in_specs=[
        pl.BlockSpec((ROWS, 1), lambda i: (i, 0)),
        pl.BlockSpec((ROWS, HALF), lambda i: (i, 0)),
        pl.BlockSpec((ROWS, HALF), lambda i: (i, 0)),
        pl.BlockSpec((ROWS, HALF), lambda i: (i, 0)),
        pl.BlockSpec((ROWS, HALF), lambda i: (i, 0)),
        pl.BlockSpec((D, DL), lambda i: (0, 0)),
        pl.BlockSpec((1, DL), lambda i: (0, 0)),
        pl.BlockSpec((D, DL), lambda i: (0, 0)),
        pl.BlockSpec((1, DL), lambda i: (0, 0)),
    ],
    out_specs=[
        pl.BlockSpec((ROWS, DL), lambda i: (i, 0)),
        pl.BlockSpec((ROWS, DL), lambda i: (i, 0)),
    ],
    out_shape=[
        jax.ShapeDtypeStruct((N, DL), jnp.float32),
        jax.ShapeDtypeStruct((N, DL), jnp.float32),
    ],
)


def kernel(Y, edge_index, W1, b1, Wmu, bmu, Wlv, blv):
    src = edge_index[0]
    dst = edge_index[1]
    dst16 = dst.reshape(NS, EPT)
    packed = jnp.bitwise_or(jnp.left_shift(src, 16), dst)
    edges_ch = packed.reshape(E // CH, CH)

    degp = _deg_kernel(dst16)
    deg_col = degp.reshape(-1)[:N].reshape(N, 1)

    p1a, p1b = _scale_call(deg_col, Y)
    a1a, a1b = _agg_kernel(edges_ch, p1a, p1b)
    p2a, p2b = _mid_call(deg_col, a1a, a1b, p1a, p1b, W1, b1.reshape(1, D))
    a2a, a2b = _agg_kernel(edges_ch, p2a, p2b)
    mu, lv = _final_call(deg_col, a2a, a2b, p2a, p2b,
                         Wmu, bmu.reshape(1, DL), Wlv, blv.reshape(1, DL))
    return (mu, lv)
